# baseline (speedup 1.0000x reference)
import functools

import jax
import jax.numpy as jnp
from jax import lax
from jax.experimental import pallas as pl
from jax.experimental.pallas import tpu as pltpu
from jax.experimental.pallas import tpu_sc as plsc

NC = 2
NS = 16
B = 128
FC = 8
NB_RING = 3


@functools.lru_cache(maxsize=None)
def _build_sc_scatter(n, n_acc, nb, e_b, f_h, f_e):
    rz = n_acc // NS

    mesh = plsc.VectorSubcoreMesh(
        core_axis_name="c", subcore_axis_name="s",
        num_cores=NC, num_subcores=NS)

    @functools.partial(
        pl.kernel,
        out_type=(
            jax.ShapeDtypeStruct((n_acc, f_h), jnp.float32),
            jax.ShapeDtypeStruct((n_acc, f_h), jnp.float32),
            jax.ShapeDtypeStruct((n_acc, f_e), jnp.float32),
            jax.ShapeDtypeStruct((n_acc, FC), jnp.float32),
        ),
        mesh=mesh,
        compiler_params=pltpu.CompilerParams(use_tc_tiling_on_sc=False),
        scratch_types=[
            pltpu.VMEM((nb, B), jnp.int32),
            pltpu.VMEM((nb, B), jnp.int32),
            pltpu.VMEM((NB_RING, B, f_h), jnp.float32),
            pltpu.VMEM((NB_RING, B, f_e), jnp.float32),
            pltpu.VMEM((B, FC), jnp.float32),
            pltpu.VMEM_SHARED((n_acc, f_h), jnp.float32),
            pltpu.VMEM_SHARED((n_acc, f_e), jnp.float32),
            pltpu.VMEM_SHARED((n_acc, FC), jnp.float32),
            pltpu.SemaphoreType.DMA((NB_RING,)),
            pltpu.SemaphoreType.DMA((NB_RING,)),
            pltpu.SemaphoreType.DMA((NB_RING,)),
            pltpu.SemaphoreType.DMA((NB_RING,)),
        ],
    )
    def sc_scatter(xh_hbm, row_hbm, col_hbm, ea_hbm, ones_hbm,
                   z64_hbm, z16_hbm, z8_hbm,
                   sx0_out, sx1_out, se_out, cnt_out,
                   row_v, col_v, xb, eb, ob, sx_sh, se_sh, cnt_sh,
                   gsem, lsem, xsem, esem):
        cid = lax.axis_index("c")
        sid = lax.axis_index("s")
        rows = pl.ds(sid * rz, rz)
        pltpu.sync_copy(z64_hbm.at[rows], sx_sh.at[rows])

        @pl.when(cid == 0)
        def _():
            pltpu.sync_copy(z16_hbm.at[rows], se_sh.at[rows])

        @pl.when(cid == 1)
        def _():
            pltpu.sync_copy(z8_hbm.at[rows], cnt_sh.at[rows])
            pltpu.sync_copy(ones_hbm, ob)

        pltpu.sync_copy(row_hbm.at[cid, sid], row_v)
        pltpu.sync_copy(col_hbm.at[sid], col_v)
        plsc.subcore_barrier()

        base0 = sid * (nb * B)

        def is_real(j):
            return base0 + j * B < e_b

        def issue_fetch(j, b):
            pltpu.async_copy(xh_hbm.at[row_v.at[j]], xb.at[b], gsem.at[b])

            @pl.when(jnp.logical_and(cid == 0, is_real(j)))
            def _():
                pltpu.async_copy(ea_hbm.at[pl.ds(base0 + j * B, B)],
                                 eb.at[b], lsem.at[b])

        issue_fetch(0, 0)
        issue_fetch(1, 1)

        def drain_scatters(j, b):
            pltpu.make_async_copy(
                xb.at[b], sx_sh.at[col_v.at[j]], xsem.at[b]).wait()

            @pl.when(jnp.logical_and(cid == 0, is_real(j)))
            def _():
                pltpu.make_async_copy(
                    eb.at[b], se_sh.at[col_v.at[j]], esem.at[b]).wait()

            @pl.when(jnp.logical_and(cid == 1, is_real(j)))
            def _():
                pltpu.make_async_copy(
                    ob, cnt_sh.at[col_v.at[j]], esem.at[b]).wait()

        def body(j0, carry):
            for b in range(NB_RING):
                j = j0 * NB_RING + b
                b2 = (b + 2) % NB_RING
                pltpu.make_async_copy(
                    xh_hbm.at[row_v.at[j]], xb.at[b], gsem.at[b]).wait()
                pltpu.async_copy(
                    xb.at[b], sx_sh.at[col_v.at[j]], xsem.at[b], add=True)

                @pl.when(jnp.logical_and(cid == 0, is_real(j)))
                def _():
                    pltpu.make_async_copy(
                        ea_hbm.at[pl.ds(base0 + j * B, B)], eb.at[b],
                        lsem.at[b]).wait()
                    pltpu.async_copy(
                        eb.at[b], se_sh.at[col_v.at[j]], esem.at[b], add=True)

                @pl.when(jnp.logical_and(cid == 1, is_real(j)))
                def _():
                    pltpu.async_copy(
                        ob, cnt_sh.at[col_v.at[j]], esem.at[b], add=True)

                @pl.when(j + 2 < nb)
                def _():
                    @pl.when(j >= 1)
                    def _():
                        drain_scatters(j - 1, b2)

                    issue_fetch(j + 2, b2)

            return carry

        lax.fori_loop(0, nb // NB_RING, body, 0)
        for b in range(NB_RING):
            drain_scatters(nb - NB_RING + b, b)
        plsc.subcore_barrier()
        @pl.when(cid == 0)
        def _():
            pltpu.sync_copy(sx_sh.at[rows], sx0_out.at[rows])
            pltpu.sync_copy(se_sh.at[rows], se_out.at[rows])

        @pl.when(cid == 1)
        def _():
            pltpu.sync_copy(sx_sh.at[rows], sx1_out.at[rows])
            pltpu.sync_copy(cnt_sh.at[rows], cnt_out.at[rows])

    return sc_scatter


def _tc_body(x_ref, sx0_ref, sx1_ref, se_ref, cnt_ref, wtx0_ref, wtx1_ref,
             wte_ref, bt_ref, wb1_ref, wb2_ref, bb_ref, out_ref):
    cnt = cnt_ref[:, 0:1]
    num = (jnp.dot(sx0_ref[...], wtx0_ref[...], preferred_element_type=jnp.float32)
           + jnp.dot(sx1_ref[...], wtx1_ref[...], preferred_element_type=jnp.float32)
           + jnp.dot(se_ref[...], wte_ref[...], preferred_element_type=jnp.float32)
           + cnt * bt_ref[...])
    mean = num / jnp.maximum(cnt, 1.0)
    z = (jnp.dot(x_ref[...], wb1_ref[...], preferred_element_type=jnp.float32)
         + jnp.dot(mean, wb2_ref[...], preferred_element_type=jnp.float32)
         + bb_ref[...])
    out_ref[...] = jnp.maximum(z, 0.0)


def kernel(x, edge_index, edge_attr, u, batch, W_top, b_top, W_bot, b_bot):
    n, f_x = x.shape
    e, f_e = edge_attr.shape
    f_h = f_x // 2
    h_top = W_top.shape[1]
    h_bot = W_bot.shape[1]

    nb = -(-e // (NS * B))
    nb = -(-nb // NB_RING) * NB_RING
    e_pad = NS * nb * B
    pad = e_pad - e
    n_acc = -(-(n + 1) // (NS * 8)) * (NS * 8)

    pad_ar = jnp.arange(pad, dtype=jnp.int32)
    row_pad = jnp.concatenate([edge_index[0], pad_ar % n])
    row_p = jnp.stack([row_pad, row_pad + n]).reshape(NC, NS, nb, B)
    col_p = jnp.concatenate(
        [edge_index[1], n + pad_ar % (n_acc - n)]).reshape(NS, nb, B)
    e_b = -(-e // B) * B
    ea_p = edge_attr if e_b == e else jnp.pad(edge_attr, ((0, e_b - e), (0, 0)))
    xh = jnp.concatenate([x[:, :f_h], x[:, f_h:]], axis=0)
    ones_b = jnp.ones((B, FC), jnp.float32)
    z64 = jnp.zeros((n_acc, f_h), jnp.float32)
    z16 = jnp.zeros((n_acc, f_e), jnp.float32)
    z8 = jnp.zeros((n_acc, FC), jnp.float32)

    sc_scatter = _build_sc_scatter(n, n_acc, nb, e_b, f_h, f_e)
    sx0, sx1, se_sum, cnt_sum = sc_scatter(
        xh, row_p, col_p, ea_p, ones_b, z64, z16, z8)

    blk = 1000
    grid = (n // blk,)
    out = pl.pallas_call(
        _tc_body,
        grid=grid,
        in_specs=[
            pl.BlockSpec((blk, f_x), lambda i: (i, 0)),
            pl.BlockSpec((blk, f_h), lambda i: (i, 0)),
            pl.BlockSpec((blk, f_h), lambda i: (i, 0)),
            pl.BlockSpec((blk, f_e), lambda i: (i, 0)),
            pl.BlockSpec((blk, FC), lambda i: (i, 0)),
            pl.BlockSpec((f_h, h_top), lambda i: (0, 0)),
            pl.BlockSpec((f_h, h_top), lambda i: (0, 0)),
            pl.BlockSpec((f_e, h_top), lambda i: (0, 0)),
            pl.BlockSpec((1, h_top), lambda i: (0, 0)),
            pl.BlockSpec((f_x, h_bot), lambda i: (0, 0)),
            pl.BlockSpec((h_top, h_bot), lambda i: (0, 0)),
            pl.BlockSpec((1, h_bot), lambda i: (0, 0)),
        ],
        out_specs=pl.BlockSpec((blk, h_bot), lambda i: (i, 0)),
        out_shape=jax.ShapeDtypeStruct((n, h_bot), jnp.float32),
    )(x, sx0, sx1, se_sum, cnt_sum,
      W_top[:f_h], W_top[f_h:f_x], W_top[f_x:], b_top[None, :],
      W_bot[:f_x], W_bot[f_x:], b_bot[None, :])
    return out


# speedup vs baseline: 1.0328x; 1.0328x over previous
import functools

import jax
import jax.numpy as jnp
from jax import lax
from jax.experimental import pallas as pl
from jax.experimental.pallas import tpu as pltpu
from jax.experimental.pallas import tpu_sc as plsc

NC = 2
NS = 16
B = 128
FC = 8
NB_RING = 3


@functools.lru_cache(maxsize=None)
def _build_sc_scatter(n, n_acc, nb, e_b, f_h, f_e):
    rz = n_acc // NS

    mesh = plsc.VectorSubcoreMesh(
        core_axis_name="c", subcore_axis_name="s",
        num_cores=NC, num_subcores=NS)

    @functools.partial(
        pl.kernel,
        out_type=(
            jax.ShapeDtypeStruct((n_acc, f_h), jnp.float32),
            jax.ShapeDtypeStruct((n_acc, f_h), jnp.float32),
        ),
        mesh=mesh,
        compiler_params=pltpu.CompilerParams(use_tc_tiling_on_sc=False),
        scratch_types=[
            pltpu.VMEM((nb, B), jnp.int32),
            pltpu.VMEM((nb, B), jnp.int32),
            pltpu.VMEM((NB_RING, B, f_h), jnp.float32),
            pltpu.VMEM_SHARED((n_acc, f_h), jnp.float32),
            pltpu.SemaphoreType.DMA((NB_RING,)),
            pltpu.SemaphoreType.DMA((NB_RING,)),
        ],
    )
    def sc_scatter_x(xh_hbm, row_hbm, col_hbm, z64_hbm,
                     sx0_out, sx1_out,
                     row_v, col_v, xb, sx_sh, gsem, xsem):
        cid = lax.axis_index("c")
        sid = lax.axis_index("s")
        rows = pl.ds(sid * rz, rz)
        pltpu.sync_copy(z64_hbm.at[rows], sx_sh.at[rows])
        pltpu.sync_copy(row_hbm.at[cid, sid], row_v)
        pltpu.sync_copy(col_hbm.at[sid], col_v)
        plsc.subcore_barrier()

        def issue_fetch(j, b):
            pltpu.async_copy(xh_hbm.at[row_v.at[j]], xb.at[b], gsem.at[b])

        issue_fetch(0, 0)
        issue_fetch(1, 1)

        def drain_scatter(j, b):
            pltpu.make_async_copy(
                xb.at[b], sx_sh.at[col_v.at[j]], xsem.at[b]).wait()

        def body(j0, carry):
            for b in range(NB_RING):
                j = j0 * NB_RING + b
                b2 = (b + 2) % NB_RING
                pltpu.make_async_copy(
                    xh_hbm.at[row_v.at[j]], xb.at[b], gsem.at[b]).wait()
                pltpu.async_copy(
                    xb.at[b], sx_sh.at[col_v.at[j]], xsem.at[b], add=True)

                @pl.when(j + 2 < nb)
                def _():
                    @pl.when(j >= 1)
                    def _():
                        drain_scatter(j - 1, b2)

                    issue_fetch(j + 2, b2)

            return carry

        lax.fori_loop(0, nb // NB_RING, body, 0)
        for b in range(NB_RING):
            drain_scatter(nb - NB_RING + b, b)
        plsc.subcore_barrier()
        @pl.when(cid == 0)
        def _():
            pltpu.sync_copy(sx_sh.at[rows], sx0_out.at[rows])

        @pl.when(cid == 1)
        def _():
            pltpu.sync_copy(sx_sh.at[rows], sx1_out.at[rows])

    return sc_scatter_x


@functools.lru_cache(maxsize=None)
def _build_sc_scatter_ea(n_acc, nbe, e_b, f_e):
    rz = n_acc // NS

    mesh = plsc.VectorSubcoreMesh(
        core_axis_name="c", subcore_axis_name="s",
        num_cores=NC, num_subcores=NS)

    @functools.partial(
        pl.kernel,
        out_type=(
            jax.ShapeDtypeStruct((NC, n_acc, f_e), jnp.float32),
            jax.ShapeDtypeStruct((NC, n_acc, FC), jnp.float32),
        ),
        mesh=mesh,
        compiler_params=pltpu.CompilerParams(use_tc_tiling_on_sc=False),
        scratch_types=[
            pltpu.VMEM((nbe, B), jnp.int32),
            pltpu.VMEM((2, B, f_e), jnp.float32),
            pltpu.VMEM((B, FC), jnp.float32),
            pltpu.VMEM_SHARED((n_acc, f_e), jnp.float32),
            pltpu.VMEM_SHARED((n_acc, FC), jnp.float32),
            pltpu.SemaphoreType.DMA((2,)),
            pltpu.SemaphoreType.DMA((2,)),
            pltpu.SemaphoreType.DMA((2,)),
        ],
    )
    def sc_scatter_ea(ea_hbm, col_hbm, ones_hbm, z16_hbm, z8_hbm,
                      se_out, cnt_out,
                      col_v, eb, ob, se_sh, cnt_sh, lsem, esem, osem):
        cid = lax.axis_index("c")
        sid = lax.axis_index("s")
        rows = pl.ds(sid * rz, rz)
        pltpu.sync_copy(z16_hbm.at[rows], se_sh.at[rows])
        pltpu.sync_copy(z8_hbm.at[rows], cnt_sh.at[rows])
        pltpu.sync_copy(ones_hbm, ob)
        pltpu.sync_copy(col_hbm.at[cid, sid], col_v)
        plsc.subcore_barrier()

        wid = cid * NS + sid
        base0 = wid * (nbe * B)

        def is_real(j):
            return base0 + j * B < e_b

        def issue_fetch(j, b):
            @pl.when(is_real(j))
            def _():
                pltpu.async_copy(ea_hbm.at[pl.ds(base0 + j * B, B)],
                                 eb.at[b], lsem.at[b])

        issue_fetch(0, 0)

        def drain_scatters(j, b):
            @pl.when(is_real(j))
            def _():
                pltpu.make_async_copy(
                    eb.at[b], se_sh.at[col_v.at[j]], esem.at[b]).wait()
                pltpu.make_async_copy(
                    ob, cnt_sh.at[col_v.at[j]], osem.at[b]).wait()

        def body(j0, carry):
            for b in range(2):
                j = j0 * 2 + b
                bo = 1 - b

                @pl.when(is_real(j))
                def _():
                    pltpu.make_async_copy(
                        ea_hbm.at[pl.ds(base0 + j * B, B)], eb.at[b],
                        lsem.at[b]).wait()
                    pltpu.async_copy(
                        eb.at[b], se_sh.at[col_v.at[j]], esem.at[b], add=True)
                    pltpu.async_copy(
                        ob, cnt_sh.at[col_v.at[j]], osem.at[b], add=True)

                @pl.when(j + 1 < nbe)
                def _():
                    @pl.when(j >= 1)
                    def _():
                        drain_scatters(j - 1, bo)

                    issue_fetch(j + 1, bo)

            return carry

        lax.fori_loop(0, nbe // 2, body, 0)
        drain_scatters(nbe - 2, 0)
        drain_scatters(nbe - 1, 1)
        plsc.subcore_barrier()
        pltpu.sync_copy(se_sh.at[rows], se_out.at[cid, rows])
        pltpu.sync_copy(cnt_sh.at[rows], cnt_out.at[cid, rows])

    return sc_scatter_ea


def _tc_body(x_ref, sx0_ref, sx1_ref, se_ref, cnt_ref, wtx0_ref, wtx1_ref,
             wte_ref, bt_ref, wb1_ref, wb2_ref, bb_ref, out_ref):
    cnt = (cnt_ref[0] + cnt_ref[1])[:, 0:1]
    se = se_ref[0] + se_ref[1]
    num = (jnp.dot(sx0_ref[...], wtx0_ref[...], preferred_element_type=jnp.float32)
           + jnp.dot(sx1_ref[...], wtx1_ref[...], preferred_element_type=jnp.float32)
           + jnp.dot(se, wte_ref[...], preferred_element_type=jnp.float32)
           + cnt * bt_ref[...])
    mean = num / jnp.maximum(cnt, 1.0)
    z = (jnp.dot(x_ref[...], wb1_ref[...], preferred_element_type=jnp.float32)
         + jnp.dot(mean, wb2_ref[...], preferred_element_type=jnp.float32)
         + bb_ref[...])
    out_ref[...] = jnp.maximum(z, 0.0)


def kernel(x, edge_index, edge_attr, u, batch, W_top, b_top, W_bot, b_bot):
    n, f_x = x.shape
    e, f_e = edge_attr.shape
    f_h = f_x // 2
    h_top = W_top.shape[1]
    h_bot = W_bot.shape[1]

    nb = -(-e // (NS * B))
    nb = -(-nb // NB_RING) * NB_RING
    e_pad = NS * nb * B
    pad = e_pad - e
    n_acc = -(-(n + 1) // (NS * 8)) * (NS * 8)

    pad_ar = jnp.arange(pad, dtype=jnp.int32)
    row_pad = jnp.concatenate([edge_index[0], pad_ar % n])
    row_p = jnp.stack([row_pad, row_pad + n]).reshape(NC, NS, nb, B)
    col_p = jnp.concatenate(
        [edge_index[1], n + pad_ar % (n_acc - n)]).reshape(NS, nb, B)
    e_b = -(-e // B) * B
    ea_p = edge_attr if e_b == e else jnp.pad(edge_attr, ((0, e_b - e), (0, 0)))
    xh = jnp.concatenate([x[:, :f_h], x[:, f_h:]], axis=0)
    ones_b = jnp.ones((B, FC), jnp.float32)
    z64 = jnp.zeros((n_acc, f_h), jnp.float32)
    z16 = jnp.zeros((n_acc, f_e), jnp.float32)
    z8 = jnp.zeros((n_acc, FC), jnp.float32)

    nbe = -(-e_b // (NC * NS * B))
    nbe += nbe % 2
    pad_e = NC * NS * nbe * B - e
    pad_er = jnp.arange(pad_e, dtype=jnp.int32)
    col_e = jnp.concatenate(
        [edge_index[1], n + pad_er % (n_acc - n)]).reshape(NC, NS, nbe, B)

    sc_scatter_x = _build_sc_scatter(n, n_acc, nb, e_b, f_h, f_e)
    sx0, sx1 = sc_scatter_x(xh, row_p, col_p, z64)
    sc_scatter_ea = _build_sc_scatter_ea(n_acc, nbe, e_b, f_e)
    se_sum, cnt_sum = sc_scatter_ea(ea_p, col_e, ones_b, z16, z8)

    blk = 1000
    grid = (n // blk,)
    out = pl.pallas_call(
        _tc_body,
        grid=grid,
        in_specs=[
            pl.BlockSpec((blk, f_x), lambda i: (i, 0)),
            pl.BlockSpec((blk, f_h), lambda i: (i, 0)),
            pl.BlockSpec((blk, f_h), lambda i: (i, 0)),
            pl.BlockSpec((NC, blk, f_e), lambda i: (0, i, 0)),
            pl.BlockSpec((NC, blk, FC), lambda i: (0, i, 0)),
            pl.BlockSpec((f_h, h_top), lambda i: (0, 0)),
            pl.BlockSpec((f_h, h_top), lambda i: (0, 0)),
            pl.BlockSpec((f_e, h_top), lambda i: (0, 0)),
            pl.BlockSpec((1, h_top), lambda i: (0, 0)),
            pl.BlockSpec((f_x, h_bot), lambda i: (0, 0)),
            pl.BlockSpec((h_top, h_bot), lambda i: (0, 0)),
            pl.BlockSpec((1, h_bot), lambda i: (0, 0)),
        ],
        out_specs=pl.BlockSpec((blk, h_bot), lambda i: (i, 0)),
        out_shape=jax.ShapeDtypeStruct((n, h_bot), jnp.float32),
    )(x, sx0, sx1, se_sum, cnt_sum,
      W_top[:f_h], W_top[f_h:f_x], W_top[f_x:], b_top[None, :],
      W_bot[:f_x], W_bot[f_x:], b_bot[None, :])
    return out
